# baseline (speedup 1.0000x reference)
import jax
import jax.numpy as jnp
from jax.experimental import pallas as pl
from jax.experimental.pallas import tpu as pltpu

COST_CLASS, COST_BBOX, COST_GIOU = 1.0, 5.0, 2.0
NUM_FRAMES, Q, T, C = 36, 100, 20, 92
GA = 6
GB = 8
NEG = -1e30


def _cost_body(logits_ref, pb_ref, lab_ref, tb_ref, cost_ref, ben_ref):
    lg = logits_ref[0]
    m = jnp.max(lg, axis=1, keepdims=True)
    e = jnp.exp(lg - m)
    s = jnp.sum(e, axis=1, keepdims=True)
    prob = e / s

    iota_c = jax.lax.broadcasted_iota(jnp.int32, (T, C), 1)

    for g in range(GA):
        prob_g = prob[Q * g:Q * (g + 1), :]
        lab_g = lab_ref[0, g]
        onehot = jnp.where(iota_c == lab_g, 1.0, 0.0)
        gathered = jax.lax.dot_general(
            onehot, prob_g, (((1,), (1,)), ((), ())),
            precision=jax.lax.Precision.HIGHEST)
        cost_class = -gathered.astype(jnp.bfloat16).astype(jnp.float32)

        pbt = jnp.transpose(pb_ref[0][Q * g:Q * (g + 1), :])
        tb = tb_ref[0, g]
        p_cx, p_cy = pbt[0:1, :], pbt[1:2, :]
        p_w, p_h = pbt[2:3, :], pbt[3:4, :]
        t_cx, t_cy = tb[:, 0:1], tb[:, 1:2]
        t_w, t_h = tb[:, 2:3], tb[:, 3:4]

        cost_bbox = (((jnp.abs(p_cx - t_cx) + jnp.abs(p_cy - t_cy))
                      + jnp.abs(p_w - t_w)) + jnp.abs(p_h - t_h))

        p_x0, p_y0 = p_cx - 0.5 * p_w, p_cy - 0.5 * p_h
        p_x1, p_y1 = p_cx + 0.5 * p_w, p_cy + 0.5 * p_h
        t_x0, t_y0 = t_cx - 0.5 * t_w, t_cy - 0.5 * t_h
        t_x1, t_y1 = t_cx + 0.5 * t_w, t_cy + 0.5 * t_h

        area_p = (p_x1 - p_x0) * (p_y1 - p_y0)
        area_t = (t_x1 - t_x0) * (t_y1 - t_y0)
        iw = jnp.clip(jnp.minimum(p_x1, t_x1) - jnp.maximum(p_x0, t_x0), 0.0)
        ih = jnp.clip(jnp.minimum(p_y1, t_y1) - jnp.maximum(p_y0, t_y0), 0.0)
        inter = iw * ih
        union = area_p + area_t - inter
        iou = inter / union
        ew = jnp.clip(jnp.maximum(p_x1, t_x1) - jnp.minimum(p_x0, t_x0), 0.0)
        eh = jnp.clip(jnp.maximum(p_y1, t_y1) - jnp.minimum(p_y0, t_y0), 0.0)
        area_e = ew * eh
        giou = iou - (area_e - union) / area_e

        cost_t = ((COST_CLASS * cost_class + COST_BBOX * cost_bbox)
                  - COST_GIOU * giou)
        cost_ref[g] = jnp.transpose(cost_t)
        ben_ref[g] = -cost_t


def _auction_body(ben_ref, pi_ref, ti_ref):
    benefit = ben_ref[...]
    smax = jnp.max(benefit, axis=(1, 2), keepdims=True)
    smin = jnp.min(benefit, axis=(1, 2), keepdims=True)
    eps = (smax - smin + 1e-6) / 1000.0

    iota_q = jax.lax.broadcasted_iota(jnp.int32, (GB, T, Q), 2)
    iota_t = jax.lax.broadcasted_iota(jnp.int32, (GB, T, Q), 1)
    imax = jnp.int32(2147483647)

    def cond(state):
        _, owner, it = state
        n_assigned = jnp.sum(jnp.where(owner >= 0, 1, 0),
                             axis=(1, 2), keepdims=True)
        return jnp.logical_and(jnp.any(n_assigned < T), it < 20000)

    def body(state):
        price, owner, it = state
        assigned_t = jnp.max(jnp.where(owner == iota_t, 1, 0),
                             axis=2, keepdims=True)
        unassigned = assigned_t == 0
        vals = benefit - price
        v1 = jnp.max(vals, axis=2, keepdims=True)
        i1 = jnp.min(jnp.where(vals == v1, iota_q, imax),
                     axis=2, keepdims=True)
        sel = iota_q == i1
        v2 = jnp.max(jnp.where(sel, NEG, vals), axis=2, keepdims=True)
        price_at = jnp.sum(jnp.where(sel, price, 0.0), axis=2, keepdims=True)
        bid = (price_at + (v1 - v2)) + eps
        bid = jnp.where(unassigned, bid, NEG)
        best_bid = jnp.max(jnp.where(sel, bid, NEG),
                           axis=1, keepdims=True)
        bb_at = jnp.sum(jnp.where(sel, best_bid, 0.0), axis=2, keepdims=True)
        win = jnp.logical_and(unassigned, bid >= bb_at)
        winner = jnp.min(jnp.where(jnp.logical_and(sel, win), iota_t, T),
                         axis=1, keepdims=True)
        has_bid = winner < T
        price = jnp.where(has_bid, best_bid, price)
        owner = jnp.where(has_bid, winner, owner)
        return price, owner, it + 1

    init = (jnp.full((GB, 1, Q), 0.0, jnp.float32),
            jnp.full((GB, 1, Q), -1, jnp.int32), jnp.int32(0))
    _, owner, _ = jax.lax.while_loop(cond, body, init)

    obj_of = jnp.min(jnp.where(owner == iota_t, iota_q, imax),
                     axis=2, keepdims=True)
    obj_of = jnp.where(obj_of == imax, -1, obj_of)

    obj_row = jnp.transpose(obj_of, (0, 2, 1))
    iota_tc = jax.lax.broadcasted_iota(jnp.int32, (GB, T, T), 1)
    iota_tr = jax.lax.broadcasted_iota(jnp.int32, (GB, T, T), 2)
    less = obj_row < obj_of
    tie = jnp.logical_and(obj_row == obj_of, iota_tr < iota_tc)
    rank = jnp.sum(jnp.where(jnp.logical_or(less, tie), 1, 0),
                   axis=2, keepdims=True)
    hit = rank == iota_tr
    pi_ref[...] = jnp.sum(jnp.where(hit, obj_of, 0), axis=1, keepdims=True)
    ti_ref[...] = jnp.sum(jnp.where(hit, iota_tc, 0), axis=1, keepdims=True)


def kernel(pred_logits, pred_boxes, tgt_labels, tgt_boxes):
    b = pred_logits.shape[0]
    n = b * NUM_FRAMES
    j_steps = NUM_FRAMES // GA
    lab4 = jnp.transpose(tgt_labels, (1, 0, 2)).reshape(b, NUM_FRAMES, T, 1)
    tb4 = jnp.transpose(tgt_boxes, (1, 0, 2, 3)).reshape(b, NUM_FRAMES, T, 4)

    cost, ben = pl.pallas_call(
        _cost_body,
        grid=(b, j_steps),
        in_specs=[
            pl.BlockSpec((1, GA * Q, C), lambda i, j: (i, j, 0)),
            pl.BlockSpec((1, GA * Q, 4), lambda i, j: (i, j, 0)),
            pl.BlockSpec((1, GA, T, 1), lambda i, j: (i, j, 0, 0)),
            pl.BlockSpec((1, GA, T, 4), lambda i, j: (i, j, 0, 0)),
        ],
        out_specs=[
            pl.BlockSpec((GA, Q, T), lambda i, j: (i * j_steps + j, 0, 0)),
            pl.BlockSpec((GA, T, Q), lambda i, j: (i * j_steps + j, 0, 0)),
        ],
        out_shape=[
            jax.ShapeDtypeStruct((n, Q, T), jnp.float32),
            jax.ShapeDtypeStruct((n, T, Q), jnp.float32),
        ],
        compiler_params=pltpu.CompilerParams(
            dimension_semantics=("parallel", "arbitrary"),
        ),
    )(pred_logits, pred_boxes, lab4, tb4)

    pred_idx, tgt_idx = pl.pallas_call(
        _auction_body,
        grid=(n // GB,),
        in_specs=[pl.BlockSpec((GB, T, Q), lambda i: (i, 0, 0))],
        out_specs=[
            pl.BlockSpec((GB, 1, T), lambda i: (i, 0, 0)),
            pl.BlockSpec((GB, 1, T), lambda i: (i, 0, 0)),
        ],
        out_shape=[
            jax.ShapeDtypeStruct((n, 1, T), jnp.int32),
            jax.ShapeDtypeStruct((n, 1, T), jnp.int32),
        ],
        compiler_params=pltpu.CompilerParams(
            dimension_semantics=("parallel",),
        ),
    )(ben)

    return (cost.reshape(b, NUM_FRAMES, Q, T),
            pred_idx.reshape(b, NUM_FRAMES, T),
            tgt_idx.reshape(b, NUM_FRAMES, T))


# speedup vs baseline: 1.1463x; 1.1463x over previous
import jax
import jax.numpy as jnp
from jax.experimental import pallas as pl
from jax.experimental.pallas import tpu as pltpu

COST_CLASS, COST_BBOX, COST_GIOU = 1.0, 5.0, 2.0
NUM_FRAMES, Q, T, C = 36, 100, 20, 92
GA = 6
GB = 8
NEG = -1e30


def _cost_body(logits_ref, pbt_ref, lab_ref, tb_ref, cost_ref, ben_ref):
    lg = logits_ref[0]
    m = jnp.max(lg, axis=1, keepdims=True)
    e = jnp.exp(lg - m)
    s = jnp.sum(e, axis=1, keepdims=True)
    prob = e / s

    iota_c = jax.lax.broadcasted_iota(jnp.int32, (T, C), 1)

    for g in range(GA):
        prob_g = prob[Q * g:Q * (g + 1), :]
        lab_g = lab_ref[0, g]
        onehot = jnp.where(iota_c == lab_g, 1.0, 0.0)
        gathered = jax.lax.dot_general(
            onehot, prob_g, (((1,), (1,)), ((), ())),
            precision=jax.lax.Precision.HIGHEST)
        cost_class = -gathered.astype(jnp.bfloat16).astype(jnp.float32)

        pbt = pbt_ref[0, g]
        tb = tb_ref[0, g]
        p_cx, p_cy = pbt[0:1, :], pbt[1:2, :]
        p_w, p_h = pbt[2:3, :], pbt[3:4, :]
        t_cx, t_cy = tb[:, 0:1], tb[:, 1:2]
        t_w, t_h = tb[:, 2:3], tb[:, 3:4]

        cost_bbox = (((jnp.abs(p_cx - t_cx) + jnp.abs(p_cy - t_cy))
                      + jnp.abs(p_w - t_w)) + jnp.abs(p_h - t_h))

        p_x0, p_y0 = p_cx - 0.5 * p_w, p_cy - 0.5 * p_h
        p_x1, p_y1 = p_cx + 0.5 * p_w, p_cy + 0.5 * p_h
        t_x0, t_y0 = t_cx - 0.5 * t_w, t_cy - 0.5 * t_h
        t_x1, t_y1 = t_cx + 0.5 * t_w, t_cy + 0.5 * t_h

        area_p = (p_x1 - p_x0) * (p_y1 - p_y0)
        area_t = (t_x1 - t_x0) * (t_y1 - t_y0)
        iw = jnp.clip(jnp.minimum(p_x1, t_x1) - jnp.maximum(p_x0, t_x0), 0.0)
        ih = jnp.clip(jnp.minimum(p_y1, t_y1) - jnp.maximum(p_y0, t_y0), 0.0)
        inter = iw * ih
        union = area_p + area_t - inter
        iou = inter / union
        ew = jnp.clip(jnp.maximum(p_x1, t_x1) - jnp.minimum(p_x0, t_x0), 0.0)
        eh = jnp.clip(jnp.maximum(p_y1, t_y1) - jnp.minimum(p_y0, t_y0), 0.0)
        area_e = ew * eh
        giou = iou - (area_e - union) / area_e

        cost_t = ((COST_CLASS * cost_class + COST_BBOX * cost_bbox)
                  - COST_GIOU * giou)
        cost_ref[g] = jnp.transpose(cost_t)
        ben_ref[g] = -cost_t


def _auction_body(ben_ref, pi_ref, ti_ref):
    benefit = ben_ref[...]
    smax = jnp.max(benefit, axis=(1, 2), keepdims=True)
    smin = jnp.min(benefit, axis=(1, 2), keepdims=True)
    eps = (smax - smin + 1e-6) / 1000.0

    iota_q = jax.lax.broadcasted_iota(jnp.int32, (GB, T, Q), 2)
    iota_t = jax.lax.broadcasted_iota(jnp.int32, (GB, T, Q), 1)
    imax = jnp.int32(2147483647)

    def cond(state):
        _, owner, it = state
        n_assigned = jnp.sum(jnp.where(owner >= 0, 1, 0),
                             axis=(1, 2), keepdims=True)
        return jnp.logical_and(jnp.any(n_assigned < T), it < 20000)

    def body(state):
        price, owner, it = state
        assigned_t = jnp.max(jnp.where(owner == iota_t, 1, 0),
                             axis=2, keepdims=True)
        unassigned = assigned_t == 0
        vals = benefit - price
        v1 = jnp.max(vals, axis=2, keepdims=True)
        i1 = jnp.min(jnp.where(vals == v1, iota_q, imax),
                     axis=2, keepdims=True)
        sel = iota_q == i1
        v2 = jnp.max(jnp.where(sel, NEG, vals), axis=2, keepdims=True)
        price_at = jnp.sum(jnp.where(sel, price, 0.0), axis=2, keepdims=True)
        bid = (price_at + (v1 - v2)) + eps
        bid = jnp.where(unassigned, bid, NEG)
        best_bid = jnp.max(jnp.where(sel, bid, NEG),
                           axis=1, keepdims=True)
        bb_at = jnp.sum(jnp.where(sel, best_bid, 0.0), axis=2, keepdims=True)
        win = jnp.logical_and(unassigned, bid >= bb_at)
        winner = jnp.min(jnp.where(jnp.logical_and(sel, win), iota_t, T),
                         axis=1, keepdims=True)
        has_bid = winner < T
        price = jnp.where(has_bid, best_bid, price)
        owner = jnp.where(has_bid, winner, owner)
        return price, owner, it + 1

    init = (jnp.full((GB, 1, Q), 0.0, jnp.float32),
            jnp.full((GB, 1, Q), -1, jnp.int32), jnp.int32(0))
    _, owner, _ = jax.lax.while_loop(cond, body, init)

    obj_of = jnp.min(jnp.where(owner == iota_t, iota_q, imax),
                     axis=2, keepdims=True)
    obj_of = jnp.where(obj_of == imax, -1, obj_of)

    obj_row = jnp.transpose(obj_of, (0, 2, 1))
    iota_tc = jax.lax.broadcasted_iota(jnp.int32, (GB, T, T), 1)
    iota_tr = jax.lax.broadcasted_iota(jnp.int32, (GB, T, T), 2)
    less = obj_row < obj_of
    tie = jnp.logical_and(obj_row == obj_of, iota_tr < iota_tc)
    rank = jnp.sum(jnp.where(jnp.logical_or(less, tie), 1, 0),
                   axis=2, keepdims=True)
    hit = rank == iota_tr
    pi_ref[...] = jnp.sum(jnp.where(hit, obj_of, 0), axis=1, keepdims=True)
    ti_ref[...] = jnp.sum(jnp.where(hit, iota_tc, 0), axis=1, keepdims=True)


def kernel(pred_logits, pred_boxes, tgt_labels, tgt_boxes):
    b = pred_logits.shape[0]
    n = b * NUM_FRAMES
    j_steps = NUM_FRAMES // GA
    pbt4 = jnp.transpose(pred_boxes.reshape(b, NUM_FRAMES, Q, 4), (0, 1, 3, 2))
    lab4 = jnp.transpose(tgt_labels, (1, 0, 2)).reshape(b, NUM_FRAMES, T, 1)
    tb4 = jnp.transpose(tgt_boxes, (1, 0, 2, 3)).reshape(b, NUM_FRAMES, T, 4)

    cost, ben = pl.pallas_call(
        _cost_body,
        grid=(b, j_steps),
        in_specs=[
            pl.BlockSpec((1, GA * Q, C), lambda i, j: (i, j, 0)),
            pl.BlockSpec((1, GA, 4, Q), lambda i, j: (i, j, 0, 0)),
            pl.BlockSpec((1, GA, T, 1), lambda i, j: (i, j, 0, 0)),
            pl.BlockSpec((1, GA, T, 4), lambda i, j: (i, j, 0, 0)),
        ],
        out_specs=[
            pl.BlockSpec((GA, Q, T), lambda i, j: (i * j_steps + j, 0, 0)),
            pl.BlockSpec((GA, T, Q), lambda i, j: (i * j_steps + j, 0, 0)),
        ],
        out_shape=[
            jax.ShapeDtypeStruct((n, Q, T), jnp.float32),
            jax.ShapeDtypeStruct((n, T, Q), jnp.float32),
        ],
        compiler_params=pltpu.CompilerParams(
            dimension_semantics=("parallel", "arbitrary"),
        ),
    )(pred_logits, pbt4, lab4, tb4)

    pred_idx, tgt_idx = pl.pallas_call(
        _auction_body,
        grid=(n // GB,),
        in_specs=[pl.BlockSpec((GB, T, Q), lambda i: (i, 0, 0))],
        out_specs=[
            pl.BlockSpec((GB, 1, T), lambda i: (i, 0, 0)),
            pl.BlockSpec((GB, 1, T), lambda i: (i, 0, 0)),
        ],
        out_shape=[
            jax.ShapeDtypeStruct((n, 1, T), jnp.int32),
            jax.ShapeDtypeStruct((n, 1, T), jnp.int32),
        ],
        compiler_params=pltpu.CompilerParams(
            dimension_semantics=("parallel",),
        ),
    )(ben)

    return (cost.reshape(b, NUM_FRAMES, Q, T),
            pred_idx.reshape(b, NUM_FRAMES, T),
            tgt_idx.reshape(b, NUM_FRAMES, T))


# speedup vs baseline: 1.1519x; 1.0049x over previous
import jax
import jax.numpy as jnp
from jax.experimental import pallas as pl
from jax.experimental.pallas import tpu as pltpu

COST_CLASS, COST_BBOX, COST_GIOU = 1.0, 5.0, 2.0
NUM_FRAMES, Q, T, C = 36, 100, 20, 92
GA = 6
GB = 8
NEG = -1e30


def _cost_body(logits_ref, pbt_ref, lab_ref, tb_ref, cost_ref, ben_ref):
    lg = logits_ref[0]
    m = jnp.max(lg, axis=1, keepdims=True)
    e = jnp.exp(lg - m)
    s = jnp.sum(e, axis=1, keepdims=True)
    prob = e / s

    iota_c = jax.lax.broadcasted_iota(jnp.int32, (C, T), 0)
    iota_8 = jax.lax.broadcasted_iota(jnp.int32, (8, T), 0)
    brow = jax.lax.rem(pl.program_id(0), 8)

    for g in range(GA):
        prob_g = prob[Q * g:Q * (g + 1), :]
        lab_row = jnp.max(jnp.where(iota_8 == brow, lab_ref[g], 0),
                          axis=0, keepdims=True)
        onehot = jnp.where(iota_c == lab_row, 1.0, 0.0)
        gathered = jax.lax.dot_general(
            onehot, prob_g, (((0,), (1,)), ((), ())),
            precision=jax.lax.Precision.HIGHEST)
        cost_class = -gathered.astype(jnp.bfloat16).astype(jnp.float32)

        pbt = pbt_ref[0, g]
        tb = tb_ref[g, 0]
        p_cx, p_cy = pbt[0:1, :], pbt[1:2, :]
        p_w, p_h = pbt[2:3, :], pbt[3:4, :]
        t_cx, t_cy = tb[:, 0:1], tb[:, 1:2]
        t_w, t_h = tb[:, 2:3], tb[:, 3:4]

        cost_bbox = (((jnp.abs(p_cx - t_cx) + jnp.abs(p_cy - t_cy))
                      + jnp.abs(p_w - t_w)) + jnp.abs(p_h - t_h))

        p_x0, p_y0 = p_cx - 0.5 * p_w, p_cy - 0.5 * p_h
        p_x1, p_y1 = p_cx + 0.5 * p_w, p_cy + 0.5 * p_h
        t_x0, t_y0 = t_cx - 0.5 * t_w, t_cy - 0.5 * t_h
        t_x1, t_y1 = t_cx + 0.5 * t_w, t_cy + 0.5 * t_h

        area_p = (p_x1 - p_x0) * (p_y1 - p_y0)
        area_t = (t_x1 - t_x0) * (t_y1 - t_y0)
        iw = jnp.clip(jnp.minimum(p_x1, t_x1) - jnp.maximum(p_x0, t_x0), 0.0)
        ih = jnp.clip(jnp.minimum(p_y1, t_y1) - jnp.maximum(p_y0, t_y0), 0.0)
        inter = iw * ih
        union = area_p + area_t - inter
        iou = inter / union
        ew = jnp.clip(jnp.maximum(p_x1, t_x1) - jnp.minimum(p_x0, t_x0), 0.0)
        eh = jnp.clip(jnp.maximum(p_y1, t_y1) - jnp.minimum(p_y0, t_y0), 0.0)
        area_e = ew * eh
        giou = iou - (area_e - union) / area_e

        cost_t = ((COST_CLASS * cost_class + COST_BBOX * cost_bbox)
                  - COST_GIOU * giou)
        cost_ref[g] = jnp.transpose(cost_t)
        ben_ref[g] = -cost_t


def _auction_body(ben_ref, pi_ref, ti_ref):
    benefit = ben_ref[...]
    smax = jnp.max(benefit, axis=(1, 2), keepdims=True)
    smin = jnp.min(benefit, axis=(1, 2), keepdims=True)
    eps = (smax - smin + 1e-6) / 1000.0

    iota_q = jax.lax.broadcasted_iota(jnp.int32, (GB, T, Q), 2)
    iota_t = jax.lax.broadcasted_iota(jnp.int32, (GB, T, Q), 1)
    imax = jnp.int32(2147483647)

    def cond(state):
        _, owner, it = state
        n_assigned = jnp.sum(jnp.where(owner >= 0, 1, 0),
                             axis=(1, 2), keepdims=True)
        return jnp.logical_and(jnp.any(n_assigned < T), it < 20000)

    def body(state):
        price, owner, it = state
        assigned_t = jnp.max(jnp.where(owner == iota_t, 1, 0),
                             axis=2, keepdims=True)
        unassigned = assigned_t == 0
        vals = benefit - price
        v1 = jnp.max(vals, axis=2, keepdims=True)
        i1 = jnp.min(jnp.where(vals == v1, iota_q, imax),
                     axis=2, keepdims=True)
        sel = iota_q == i1
        v2 = jnp.max(jnp.where(sel, NEG, vals), axis=2, keepdims=True)
        price_at = jnp.sum(jnp.where(sel, price, 0.0), axis=2, keepdims=True)
        bid = (price_at + (v1 - v2)) + eps
        bid = jnp.where(unassigned, bid, NEG)
        best_bid = jnp.max(jnp.where(sel, bid, NEG),
                           axis=1, keepdims=True)
        bb_at = jnp.sum(jnp.where(sel, best_bid, 0.0), axis=2, keepdims=True)
        win = jnp.logical_and(unassigned, bid >= bb_at)
        winner = jnp.min(jnp.where(jnp.logical_and(sel, win), iota_t, T),
                         axis=1, keepdims=True)
        has_bid = winner < T
        price = jnp.where(has_bid, best_bid, price)
        owner = jnp.where(has_bid, winner, owner)
        return price, owner, it + 1

    init = (jnp.full((GB, 1, Q), 0.0, jnp.float32),
            jnp.full((GB, 1, Q), -1, jnp.int32), jnp.int32(0))
    _, owner, _ = jax.lax.while_loop(cond, body, init)

    obj_of = jnp.min(jnp.where(owner == iota_t, iota_q, imax),
                     axis=2, keepdims=True)
    obj_of = jnp.where(obj_of == imax, -1, obj_of)

    obj_row = jnp.transpose(obj_of, (0, 2, 1))
    iota_tc = jax.lax.broadcasted_iota(jnp.int32, (GB, T, T), 1)
    iota_tr = jax.lax.broadcasted_iota(jnp.int32, (GB, T, T), 2)
    less = obj_row < obj_of
    tie = jnp.logical_and(obj_row == obj_of, iota_tr < iota_tc)
    rank = jnp.sum(jnp.where(jnp.logical_or(less, tie), 1, 0),
                   axis=2, keepdims=True)
    hit = rank == iota_tr
    pi_ref[...] = jnp.sum(jnp.where(hit, obj_of, 0), axis=1, keepdims=True)
    ti_ref[...] = jnp.sum(jnp.where(hit, iota_tc, 0), axis=1, keepdims=True)


def kernel(pred_logits, pred_boxes, tgt_labels, tgt_boxes):
    b = pred_logits.shape[0]
    n = b * NUM_FRAMES
    j_steps = NUM_FRAMES // GA
    pbt4 = jnp.transpose(pred_boxes.reshape(b, NUM_FRAMES, Q, 4), (0, 1, 3, 2))

    cost, ben = pl.pallas_call(
        _cost_body,
        grid=(b, j_steps),
        in_specs=[
            pl.BlockSpec((1, GA * Q, C), lambda i, j: (i, j, 0)),
            pl.BlockSpec((1, GA, 4, Q), lambda i, j: (i, j, 0, 0)),
            pl.BlockSpec((GA, 8, T), lambda i, j: (j, i // 8, 0)),
            pl.BlockSpec((GA, 1, T, 4), lambda i, j: (j, i, 0, 0)),
        ],
        out_specs=[
            pl.BlockSpec((GA, Q, T), lambda i, j: (i * j_steps + j, 0, 0)),
            pl.BlockSpec((GA, T, Q), lambda i, j: (i * j_steps + j, 0, 0)),
        ],
        out_shape=[
            jax.ShapeDtypeStruct((n, Q, T), jnp.float32),
            jax.ShapeDtypeStruct((n, T, Q), jnp.float32),
        ],
        compiler_params=pltpu.CompilerParams(
            dimension_semantics=("parallel", "arbitrary"),
        ),
    )(pred_logits, pbt4, tgt_labels, tgt_boxes)

    pred_idx, tgt_idx = pl.pallas_call(
        _auction_body,
        grid=(n // GB,),
        in_specs=[pl.BlockSpec((GB, T, Q), lambda i: (i, 0, 0))],
        out_specs=[
            pl.BlockSpec((GB, 1, T), lambda i: (i, 0, 0)),
            pl.BlockSpec((GB, 1, T), lambda i: (i, 0, 0)),
        ],
        out_shape=[
            jax.ShapeDtypeStruct((n, 1, T), jnp.int32),
            jax.ShapeDtypeStruct((n, 1, T), jnp.int32),
        ],
        compiler_params=pltpu.CompilerParams(
            dimension_semantics=("parallel",),
        ),
    )(ben)

    return (cost.reshape(b, NUM_FRAMES, Q, T),
            pred_idx.reshape(b, NUM_FRAMES, T),
            tgt_idx.reshape(b, NUM_FRAMES, T))
